# baseline (speedup 1.0000x reference)
import functools

import jax
import jax.numpy as jnp
from jax import lax
from jax.experimental import pallas as pl
from jax.experimental.pallas import tpu as pltpu
from jax.experimental.pallas import tpu_sc as plsc

V_TOTAL = 1024
N_FIXED = 256
D = 4096
B = 16 * 1024
_MERGE_BLK = 128


def _merge_body(e_ref, t_ref, o_ref):
    i = pl.program_id(0)
    nfix = N_FIXED // _MERGE_BLK

    @pl.when(i < nfix)
    def _():
        o_ref[...] = e_ref[...]

    @pl.when(i >= nfix)
    def _():
        o_ref[...] = t_ref[...]


def _build_merged(embeddings_weight, trainable_weight):
    nfix = N_FIXED // _MERGE_BLK
    return pl.pallas_call(
        _merge_body,
        grid=(V_TOTAL // _MERGE_BLK,),
        in_specs=[
            pl.BlockSpec((_MERGE_BLK, D), lambda i: (jnp.minimum(i, nfix - 1), 0)),
            pl.BlockSpec((_MERGE_BLK, D), lambda i: (jnp.maximum(i - nfix, 0), 0)),
        ],
        out_specs=pl.BlockSpec((_MERGE_BLK, D), lambda i: (i, 0)),
        out_shape=jax.ShapeDtypeStruct((V_TOTAL, D), jnp.float32),
    )(embeddings_weight, trainable_weight)


def _make_gather(nw, nc, bpw, k, nbuf):
    nchunk = bpw // k
    ngroup = -(-nchunk // nbuf)
    mesh = plsc.VectorSubcoreMesh(core_axis_name="c", subcore_axis_name="s")

    @functools.partial(
        pl.kernel,
        mesh=mesh,
        out_type=jax.ShapeDtypeStruct((B, D), jnp.float32),
        scratch_types=[
            pltpu.VMEM((nchunk, k), jnp.int32),
            pltpu.VMEM((nbuf, k, D), jnp.float32),
        ]
        + [pltpu.SemaphoreType.DMA] * (2 * nbuf),
    )
    def gather(table_hbm, idx_hbm, out_hbm, idx_v, buf_v, *sems):
        gsems, ssems = sems[:nbuf], sems[nbuf:]
        wid = lax.axis_index("s") * nc + lax.axis_index("c")
        base = wid * bpw
        pltpu.sync_copy(idx_hbm.at[wid], idx_v)

        def gather_chunk(c, b):
            return pltpu.make_async_copy(
                table_hbm.at[idx_v.at[c]], buf_v.at[b], gsems[b])

        def write_chunk(c, b):
            return pltpu.make_async_copy(
                buf_v.at[b], out_hbm.at[pl.ds(base + c * k, k)], ssems[b])

        for c in range(nbuf - 1):
            gather_chunk(c, c % nbuf).start()

        def group(g, carry):
            c0 = g * nbuf
            for b in range(nbuf):
                c = c0 + b
                la = c + nbuf - 1
                slot_la = (b + nbuf - 1) % nbuf

                @pl.when(jnp.logical_and(c >= 1, la < nchunk))
                def _():
                    write_chunk(c - 1, slot_la).wait()
                    gather_chunk(la, slot_la).start()

                @pl.when(jnp.logical_and(c < 1, la < nchunk))
                def _():
                    gather_chunk(la, slot_la).start()

                @pl.when(c < nchunk)
                def _():
                    gather_chunk(c, b).wait()
                    write_chunk(c, b).start()
            return carry

        lax.fori_loop(0, ngroup, group, 0)
        for t in range(nbuf):
            c = nchunk - nbuf + t
            write_chunk(c, c % nbuf).wait()

    return gather


def _probe_tc_copy(table):
    def body(t_ref, o_ref):
        o_ref[...] = t_ref[...] * 2.0

    return pl.pallas_call(
        body,
        grid=(32,),
        in_specs=[pl.BlockSpec((_MERGE_BLK, D), lambda i: (i % 8, 0))],
        out_specs=pl.BlockSpec((_MERGE_BLK, D), lambda i: (i, 0)),
        out_shape=jax.ShapeDtypeStruct((32 * _MERGE_BLK, D), jnp.float32),
    )(table)


def kernel(indices, embeddings_weight, trainable_weight):
    info = plsc.get_sparse_core_info()
    nc, ns = info.num_cores, info.num_subcores
    nw = nc * ns
    bpw = B // nw
    k = 8
    nbuf = 3

    merged = _build_merged(embeddings_weight, trainable_weight)
    idx = indices.astype(jnp.int32).reshape(nw, bpw // k, k)
    out = _make_gather(nw, nc, bpw, k, nbuf)(merged, idx)
    dummy = _probe_tc_copy(embeddings_weight)
    out = out.at[0, 0].add(jnp.where(dummy[0, 0] > jnp.inf, 1.0, 0.0))
    return out.reshape(indices.shape[0], indices.shape[1], D)


# speedup vs baseline: 1.1473x; 1.1473x over previous
import functools

import jax
import jax.numpy as jnp
from jax import lax
from jax.experimental import pallas as pl
from jax.experimental.pallas import tpu as pltpu
from jax.experimental.pallas import tpu_sc as plsc

V_TOTAL = 1024
N_FIXED = 256
D = 4096
B = 16 * 1024
_MERGE_BLK = 128


def _merge_body(e_ref, t_ref, o_ref):
    i = pl.program_id(0)
    nfix = N_FIXED // _MERGE_BLK

    @pl.when(i < nfix)
    def _():
        o_ref[...] = e_ref[...]

    @pl.when(i >= nfix)
    def _():
        o_ref[...] = t_ref[...]


def _build_merged(embeddings_weight, trainable_weight):
    nfix = N_FIXED // _MERGE_BLK
    return pl.pallas_call(
        _merge_body,
        grid=(V_TOTAL // _MERGE_BLK,),
        in_specs=[
            pl.BlockSpec((_MERGE_BLK, D), lambda i: (jnp.minimum(i, nfix - 1), 0)),
            pl.BlockSpec((_MERGE_BLK, D), lambda i: (jnp.maximum(i - nfix, 0), 0)),
        ],
        out_specs=pl.BlockSpec((_MERGE_BLK, D), lambda i: (i, 0)),
        out_shape=jax.ShapeDtypeStruct((V_TOTAL, D), jnp.float32),
    )(embeddings_weight, trainable_weight)


def _make_gather(nw, nc, bpw, k, nbuf):
    nchunk = bpw // k
    ngroup = -(-nchunk // nbuf)
    mesh = plsc.VectorSubcoreMesh(core_axis_name="c", subcore_axis_name="s")

    @functools.partial(
        pl.kernel,
        mesh=mesh,
        out_type=jax.ShapeDtypeStruct((B, D), jnp.float32),
        scratch_types=[
            pltpu.VMEM((nchunk, k), jnp.int32),
            pltpu.VMEM((nbuf, k, D), jnp.float32),
        ]
        + [pltpu.SemaphoreType.DMA] * (2 * nbuf),
    )
    def gather(table_hbm, idx_hbm, out_hbm, idx_v, buf_v, *sems):
        gsems, ssems = sems[:nbuf], sems[nbuf:]
        wid = lax.axis_index("s") * nc + lax.axis_index("c")
        base = wid * bpw
        pltpu.sync_copy(idx_hbm.at[wid], idx_v)

        def gather_chunk(c, b):
            return pltpu.make_async_copy(
                table_hbm.at[idx_v.at[c]], buf_v.at[b], gsems[b])

        def write_chunk(c, b):
            return pltpu.make_async_copy(
                buf_v.at[b], out_hbm.at[pl.ds(base + c * k, k)], ssems[b])

        for b in range(nbuf):
            gather_chunk(b, b).start()

        def group(g, carry):
            c0 = g * nbuf
            for b in range(nbuf):
                @pl.when(c0 + b < nchunk)
                def _():
                    gather_chunk(c0 + b, b).wait()
            for b in range(nbuf):
                @pl.when(c0 + b < nchunk)
                def _():
                    write_chunk(c0 + b, b).start()
            for b in range(nbuf):
                @pl.when(c0 + b < nchunk)
                def _():
                    write_chunk(c0 + b, b).wait()
            for b in range(nbuf):
                @pl.when(c0 + nbuf + b < nchunk)
                def _():
                    gather_chunk(c0 + nbuf + b, b).start()
            return carry

        lax.fori_loop(0, ngroup, group, 0)

    return gather


def _probe_tc_copy(table):
    def body(t_ref, o_ref):
        o_ref[...] = t_ref[...] * 2.0

    return pl.pallas_call(
        body,
        grid=(32,),
        in_specs=[pl.BlockSpec((_MERGE_BLK, D), lambda i: (i % 8, 0))],
        out_specs=pl.BlockSpec((_MERGE_BLK, D), lambda i: (i, 0)),
        out_shape=jax.ShapeDtypeStruct((32 * _MERGE_BLK, D), jnp.float32),
    )(table)


def kernel(indices, embeddings_weight, trainable_weight):
    info = plsc.get_sparse_core_info()
    nc, ns = info.num_cores, info.num_subcores
    nw = nc * ns
    bpw = B // nw
    k = 8
    nbuf = 3

    merged = _build_merged(embeddings_weight, trainable_weight)
    idx = indices.astype(jnp.int32).reshape(nw, bpw // k, k)
    out = _make_gather(nw, nc, bpw, k, nbuf)(merged, idx)
    return out.reshape(indices.shape[0], indices.shape[1], D)


# speedup vs baseline: 1.1947x; 1.0413x over previous
import functools

import jax
import jax.numpy as jnp
from jax import lax
from jax.experimental import pallas as pl
from jax.experimental.pallas import tpu as pltpu
from jax.experimental.pallas import tpu_sc as plsc

V_TOTAL = 1024
N_FIXED = 256
D = 4096
B = 16 * 1024
_MERGE_BLK = 128


def _merge_body(e_ref, t_ref, o_ref):
    i = pl.program_id(0)
    nfix = N_FIXED // _MERGE_BLK

    @pl.when(i < nfix)
    def _():
        o_ref[...] = e_ref[...]

    @pl.when(i >= nfix)
    def _():
        o_ref[...] = t_ref[...]


def _build_merged(embeddings_weight, trainable_weight):
    nfix = N_FIXED // _MERGE_BLK
    return pl.pallas_call(
        _merge_body,
        grid=(V_TOTAL // _MERGE_BLK,),
        in_specs=[
            pl.BlockSpec((_MERGE_BLK, D), lambda i: (jnp.minimum(i, nfix - 1), 0)),
            pl.BlockSpec((_MERGE_BLK, D), lambda i: (jnp.maximum(i - nfix, 0), 0)),
        ],
        out_specs=pl.BlockSpec((_MERGE_BLK, D), lambda i: (i, 0)),
        out_shape=jax.ShapeDtypeStruct((V_TOTAL, D), jnp.float32),
    )(embeddings_weight, trainable_weight)


def _make_gather(nw, nc, bpw, k, nbuf):
    nchunk = bpw // k
    ngroup = -(-nchunk // nbuf)
    mesh = plsc.VectorSubcoreMesh(core_axis_name="c", subcore_axis_name="s")

    @functools.partial(
        pl.kernel,
        mesh=mesh,
        out_type=jax.ShapeDtypeStruct((B, D), jnp.float32),
        scratch_types=[
            pltpu.VMEM((nchunk, k), jnp.int32),
            pltpu.VMEM((nbuf, k, D), jnp.float32),
        ]
        + [pltpu.SemaphoreType.DMA] * (2 * nbuf),
    )
    def gather(table_hbm, idx_hbm, out_hbm, idx_v, buf_v, *sems):
        gsems, ssems = sems[:nbuf], sems[nbuf:]
        wid = lax.axis_index("s") * nc + lax.axis_index("c")
        base = wid * bpw
        pltpu.sync_copy(idx_hbm.at[wid], idx_v)

        def gather_chunk(c, b):
            return pltpu.make_async_copy(
                table_hbm.at[idx_v.at[c]], buf_v.at[b], gsems[b])

        def write_chunk(c, b):
            return pltpu.make_async_copy(
                buf_v.at[b], out_hbm.at[pl.ds(base + c * k, k)], ssems[b])

        for c in range(nbuf - 1):
            gather_chunk(c, c % nbuf).start()

        def group(g, carry):
            c0 = g * nbuf
            for b in range(nbuf):
                c = c0 + b
                la = c + nbuf - 1
                slot_la = (b + nbuf - 1) % nbuf

                @pl.when(jnp.logical_and(c >= 1, la < nchunk))
                def _():
                    write_chunk(c - 1, slot_la).wait()
                    gather_chunk(la, slot_la).start()

                @pl.when(jnp.logical_and(c < 1, la < nchunk))
                def _():
                    gather_chunk(la, slot_la).start()

                @pl.when(c < nchunk)
                def _():
                    gather_chunk(c, b).wait()
                    write_chunk(c, b).start()
            return carry

        lax.fori_loop(0, ngroup, group, 0)
        for t in range(nbuf):
            c = nchunk - nbuf + t
            write_chunk(c, c % nbuf).wait()

    return gather


def kernel(indices, embeddings_weight, trainable_weight):
    info = plsc.get_sparse_core_info()
    nc, ns = info.num_cores, info.num_subcores
    nw = nc * ns
    bpw = B // nw
    k = 8
    nbuf = 3

    merged = _build_merged(embeddings_weight, trainable_weight)
    idx = indices.astype(jnp.int32).reshape(nw, bpw // k, k)
    out = _make_gather(nw, nc, bpw, k, nbuf)(merged, idx)
    return out.reshape(indices.shape[0], indices.shape[1], D)
